# Initial kernel scaffold; baseline (speedup 1.0000x reference)
import jax
import jax.numpy as jnp
from jax.experimental import pallas as pl


def kernel(x, edge_index, edge_attr, W_ih0, W_hh0, b_ih0, b_hh0, W_ih1, W_hh1, b_ih1, b_hh1, W_gat, att_src, att_dst, W_edge, att_edge, bias_gat, W_fc, b_fc):
    raise NotImplementedError("write your pallas kernel here")


# speedup vs baseline: 1.1681x; 1.1681x over previous
import functools

import jax
import jax.numpy as jnp
from jax import lax
from jax.experimental import pallas as pl
from jax.experimental.pallas import tpu as pltpu
from jax.experimental.pallas import tpu_sc as plsc

NS = 16
T = 32
LH = 32
HEADS = 4
C = 16
NC = 8

_INTERPRET = False

BLK_N = 2000
BLK_E = 2000


def _sig(x):
    return 0.5 * jnp.tanh(0.5 * x) + 0.5


def _lrelu(x):
    return jnp.maximum(x, 0.0) + 0.2 * jnp.minimum(x, 0.0)


def _k1_body(ea_ref, mean_ref):
    mean_ref[...] = jnp.full((1, 128), jnp.sum(ea_ref[...]), jnp.float32)


def _k1(ea_flat2d):
    rows = ea_flat2d.shape[0]
    s = pl.pallas_call(
        _k1_body,
        grid=(1,),
        in_specs=[pl.BlockSpec((rows, 128), lambda i: (0, 0))],
        out_specs=pl.BlockSpec((1, 128), lambda i: (0, 0)),
        out_shape=jax.ShapeDtypeStruct((1, 128), jnp.float32),
        interpret=_INTERPRET,
    )(ea_flat2d)
    return s[0, 0]


def _k2_body(x_ref, wih0_ref, whh0_ref, b0_ref, wih1_ref, whh1_ref, b1_ref,
             wgat_ref, asrcw_ref, adstw_ref, awself_ref,
             xp_ref, asrc_ref, adst_ref, aself_ref):
    xb = x_ref[...]
    bsz = xb.shape[0]
    static = xb[:, :NS]
    wih0 = wih0_ref[...]
    whh0 = whh0_ref[...]
    b0 = b0_ref[...]
    wih1 = wih1_ref[...]
    whh1 = whh1_ref[...]
    b1 = b1_ref[...]
    z = jnp.zeros((bsz, LH), jnp.float32)
    h0, c0, h1, c1 = z, z, z, z
    for t in range(T):
        xt = xb[:, NS + t][:, None]
        g0 = xt * wih0 + jnp.dot(h0, whh0, preferred_element_type=jnp.float32) + b0
        i0 = _sig(g0[:, 0:LH])
        f0 = _sig(g0[:, LH:2 * LH])
        gg0 = jnp.tanh(g0[:, 2 * LH:3 * LH])
        o0 = _sig(g0[:, 3 * LH:4 * LH])
        c0 = f0 * c0 + i0 * gg0
        h0 = o0 * jnp.tanh(c0)
        g1 = (jnp.dot(h0, wih1, preferred_element_type=jnp.float32)
              + jnp.dot(h1, whh1, preferred_element_type=jnp.float32) + b1)
        i1 = _sig(g1[:, 0:LH])
        f1 = _sig(g1[:, LH:2 * LH])
        gg1 = jnp.tanh(g1[:, 2 * LH:3 * LH])
        o1 = _sig(g1[:, 3 * LH:4 * LH])
        c1 = f1 * c1 + i1 * gg1
        h1 = o1 * jnp.tanh(c1)
    comb = jnp.concatenate([h1, static], axis=1)
    xp = jnp.dot(comb, wgat_ref[...], preferred_element_type=jnp.float32)
    asrc = jnp.dot(xp, asrcw_ref[...], preferred_element_type=jnp.float32)
    adst = jnp.dot(xp, adstw_ref[...], preferred_element_type=jnp.float32)
    xp_ref[...] = jnp.transpose(xp.reshape(bsz, HEADS, C), (1, 0, 2))
    asrc_ref[...] = asrc
    adst_ref[...] = adst
    aself_ref[...] = _lrelu(asrc + adst + awself_ref[...])


def _k2(x, wih0, whh0, b0, wih1, whh1, b1, wgat, asrcw, adstw, awself):
    n = x.shape[0]
    grid = n // BLK_N
    full = lambda *s: pl.BlockSpec(s, lambda i: tuple(0 for _ in s))
    return pl.pallas_call(
        _k2_body,
        grid=(grid,),
        in_specs=[pl.BlockSpec((BLK_N, NS + T), lambda i: (i, 0)),
                  full(1, 128), full(LH, 128), full(1, 128),
                  full(LH, 128), full(LH, 128), full(1, 128),
                  full(NS + LH, HEADS * C),
                  full(HEADS * C, HEADS), full(HEADS * C, HEADS),
                  full(1, HEADS)],
        out_specs=[pl.BlockSpec((HEADS, BLK_N, C), lambda i: (0, i, 0)),
                   pl.BlockSpec((BLK_N, HEADS), lambda i: (i, 0)),
                   pl.BlockSpec((BLK_N, HEADS), lambda i: (i, 0)),
                   pl.BlockSpec((BLK_N, HEADS), lambda i: (i, 0))],
        out_shape=[jax.ShapeDtypeStruct((HEADS, n, C), jnp.float32),
                   jax.ShapeDtypeStruct((n, HEADS), jnp.float32),
                   jax.ShapeDtypeStruct((n, HEADS), jnp.float32),
                   jax.ShapeDtypeStruct((n, HEADS), jnp.float32)],
        interpret=_INTERPRET,
    )(x, wih0, whh0, b0, wih1, whh1, b1, wgat, asrcw, adstw, awself)


def _k4_body(ga_ref, gd_ref, ea4_ref, wtile_ref, t_ref):
    t_ref[...] = jnp.exp(_lrelu(
        ga_ref[...] + gd_ref[...] + ea4_ref[...] * wtile_ref[...]))


def _k4(ga, gd, ea4, wvec):
    rows = ga.shape[0]
    grid = rows // BLK_E
    spec = pl.BlockSpec((BLK_E, 128), lambda i: (i, 0))
    wtile = jnp.tile(wvec, 32)[None, :]
    return pl.pallas_call(
        _k4_body,
        grid=(grid,),
        in_specs=[spec, spec, spec, pl.BlockSpec((1, 128), lambda i: (0, 0))],
        out_specs=spec,
        out_shape=jax.ShapeDtypeStruct((rows, 128), jnp.float32),
        interpret=_INTERPRET,
    )(ga, gd, ea4, wtile)


def _k7_body(agg_ref, den_ref, aself_ref, xp_ref, bias_ref, wfc_ref, bfc_ref,
             out_ref):
    bsz = aself_ref.shape[0]
    agg = jnp.transpose(agg_ref[...], (1, 0, 2)).reshape(bsz, HEADS * C)
    xp = jnp.transpose(xp_ref[...], (1, 0, 2)).reshape(bsz, HEADS * C)
    q = jnp.exp(aself_ref[...])
    den = den_ref[0] + den_ref[1] + q
    q64 = jnp.broadcast_to(q[:, :, None], (bsz, HEADS, C)).reshape(bsz, HEADS * C)
    d64 = jnp.broadcast_to(den[:, :, None], (bsz, HEADS, C)).reshape(bsz, HEADS * C)
    o = (agg + q64 * xp) / d64 + bias_ref[...]
    o = jnp.where(o > 0, o, jnp.exp(jnp.minimum(o, 0.0)) - 1.0)
    o = jnp.dot(o, wfc_ref[...], preferred_element_type=jnp.float32) + bfc_ref[...]
    out_ref[...] = jnp.maximum(o, 0.0)


def _k7(aggp, den4p, aself4, xp_hm, bias, wfc, bfc):
    n = aself4.shape[0]
    grid = n // BLK_N
    full = lambda *s: pl.BlockSpec(s, lambda i: tuple(0 for _ in s))
    return pl.pallas_call(
        _k7_body,
        grid=(grid,),
        in_specs=[pl.BlockSpec((HEADS, BLK_N, C), lambda i: (0, i, 0)),
                  pl.BlockSpec((2, BLK_N, HEADS), lambda i: (0, i, 0)),
                  pl.BlockSpec((BLK_N, HEADS), lambda i: (i, 0)),
                  pl.BlockSpec((HEADS, BLK_N, C), lambda i: (0, i, 0)),
                  full(1, HEADS * C), full(HEADS * C, NC), full(1, NC)],
        out_specs=pl.BlockSpec((BLK_N, NC), lambda i: (i, 0)),
        out_shape=jax.ShapeDtypeStruct((n, NC), jnp.float32),
        interpret=_INTERPRET,
    )(aggp, den4p, aself4, xp_hm, bias, wfc, bfc)


def kernel(x, edge_index, edge_attr,
           W_ih0, W_hh0, b_ih0, b_hh0,
           W_ih1, W_hh1, b_ih1, b_hh1,
           W_gat, att_src, att_dst, W_edge, att_edge, bias_gat,
           W_fc, b_fc):
    n = x.shape[0]
    e = edge_attr.shape[0]
    src = edge_index[0].astype(jnp.int32)
    dst = edge_index[1].astype(jnp.int32)

    b0 = (b_ih0 + b_hh0)[None, :]
    b1 = (b_ih1 + b_hh1)[None, :]
    wih0 = W_ih0.T
    whh0 = W_hh0.T
    wih1 = W_ih1.T
    whh1 = W_hh1.T
    hm = (jnp.arange(HEADS * C) // C)[:, None] == jnp.arange(HEADS)[None, :]
    adstw = jnp.where(hm, att_dst.reshape(-1)[:, None], 0.0)
    asrcw = jnp.where(hm, att_src.reshape(-1)[:, None], 0.0)
    wvec = jnp.sum(W_edge.reshape(HEADS, C) * att_edge, axis=1)

    ea2d = edge_attr.reshape(e // 128, 128)
    ea_sum = _k1(ea2d)
    mean_ea = ea_sum / e
    awself = (mean_ea * wvec)[None, :]

    xp_hm, asrc4, adst4, aself4 = _k2(
        x, wih0, whh0, b0, wih1, whh1, b1, W_gat, asrcw, adstw, awself)

    ga = asrc4[src].reshape(e * 4 // 128, 128)
    gd = adst4[dst].reshape(e * 4 // 128, 128)
    ea4 = jnp.broadcast_to(edge_attr, (e, HEADS)).reshape(e * 4 // 128, 128)
    t4_flat = _k4(ga, gd, ea4, wvec)
    t4 = t4_flat.reshape(e, HEADS)
    den4 = jax.ops.segment_sum(t4, dst, num_segments=n)
    den4p = jnp.stack([den4, jnp.zeros_like(den4)])
    xp_nm = jnp.transpose(xp_hm, (1, 0, 2))
    msg = xp_nm[src] * t4[:, :, None]
    aggp = jnp.transpose(
        jax.ops.segment_sum(msg, dst, num_segments=n), (1, 0, 2))

    out = _k7(aggp, den4p, aself4, xp_hm,
              bias_gat[None, :], W_fc, b_fc[None, :])
    return out


# speedup vs baseline: 22.2379x; 19.0378x over previous
import functools

import jax
import jax.numpy as jnp
from jax import lax
from jax.experimental import pallas as pl
from jax.experimental.pallas import tpu as pltpu
from jax.experimental.pallas import tpu_sc as plsc

NS = 16
T = 32
LH = 32
HEADS = 4
C = 16
NC = 8

BLK_N = 2000
BLK_K4 = 1000

CH3 = 2560
CH5 = 1280
CH6 = 640
NROW = 800


def _sig(x):
    return 0.5 * jnp.tanh(0.5 * x) + 0.5


def _lrelu(x):
    return jnp.maximum(x, 0.0) + 0.2 * jnp.minimum(x, 0.0)


def _sc_mesh():
    return plsc.VectorSubcoreMesh(core_axis_name="c", subcore_axis_name="s")


_SC_PARAMS = dict(
    compiler_params=pltpu.CompilerParams(use_tc_tiling_on_sc=False))


def _k1_body(ea_ref, mean_ref):
    mean_ref[...] = jnp.full((1, 128), jnp.sum(ea_ref[...]), jnp.float32)


def _k1(ea_flat2d):
    rows = ea_flat2d.shape[0]
    s = pl.pallas_call(
        _k1_body,
        grid=(1,),
        in_specs=[pl.BlockSpec((rows, 128), lambda i: (0, 0))],
        out_specs=pl.BlockSpec((1, 128), lambda i: (0, 0)),
        out_shape=jax.ShapeDtypeStruct((1, 128), jnp.float32),
    )(ea_flat2d)
    return s[0, 0]


def _k2_body(x_ref, wih0_ref, whh0_ref, b0_ref, wih1_ref, whh1_ref, b1_ref,
             wgat_ref, asrcw_ref, adstw_ref, awself_ref,
             xp_ref, tabs_ref, tabd_ref, aself_ref):
    xb = x_ref[...]
    bsz = xb.shape[0]
    static = xb[:, :NS]
    wih0 = wih0_ref[...]
    whh0 = whh0_ref[...]
    b0 = b0_ref[...]
    wih1 = wih1_ref[...]
    whh1 = whh1_ref[...]
    b1 = b1_ref[...]
    z = jnp.zeros((bsz, LH), jnp.float32)
    h0, c0, h1, c1 = z, z, z, z
    for t in range(T):
        xt = xb[:, NS + t][:, None]
        g0 = xt * wih0 + jnp.dot(h0, whh0, preferred_element_type=jnp.float32) + b0
        i0 = _sig(g0[:, 0:LH])
        f0 = _sig(g0[:, LH:2 * LH])
        gg0 = jnp.tanh(g0[:, 2 * LH:3 * LH])
        o0 = _sig(g0[:, 3 * LH:4 * LH])
        c0 = f0 * c0 + i0 * gg0
        h0 = o0 * jnp.tanh(c0)
        g1 = (jnp.dot(h0, wih1, preferred_element_type=jnp.float32)
              + jnp.dot(h1, whh1, preferred_element_type=jnp.float32) + b1)
        i1 = _sig(g1[:, 0:LH])
        f1 = _sig(g1[:, LH:2 * LH])
        gg1 = jnp.tanh(g1[:, 2 * LH:3 * LH])
        o1 = _sig(g1[:, 3 * LH:4 * LH])
        c1 = f1 * c1 + i1 * gg1
        h1 = o1 * jnp.tanh(c1)
    comb = jnp.concatenate([h1, static], axis=1)
    xp = jnp.dot(comb, wgat_ref[...], preferred_element_type=jnp.float32)
    asrc = jnp.dot(xp, asrcw_ref[...], preferred_element_type=jnp.float32)
    adst = jnp.dot(xp, adstw_ref[...], preferred_element_type=jnp.float32)
    pad = jnp.zeros((bsz, C - HEADS), jnp.float32)
    xp_ref[...] = jnp.transpose(xp.reshape(bsz, HEADS, C), (1, 0, 2))
    tabs_ref[...] = jnp.concatenate([asrc, pad], axis=1)
    tabd_ref[...] = jnp.concatenate([adst, pad], axis=1)
    aself_ref[...] = _lrelu(asrc + adst + awself_ref[...])


def _k2(x, wih0, whh0, b0, wih1, whh1, b1, wgat, asrcw, adstw, awself):
    n = x.shape[0]
    grid = n // BLK_N
    full = lambda *s: pl.BlockSpec(s, lambda i: tuple(0 for _ in s))
    return pl.pallas_call(
        _k2_body,
        grid=(grid,),
        in_specs=[pl.BlockSpec((BLK_N, NS + T), lambda i: (i, 0)),
                  full(1, 128), full(LH, 128), full(1, 128),
                  full(LH, 128), full(LH, 128), full(1, 128),
                  full(NS + LH, HEADS * C),
                  full(HEADS * C, HEADS), full(HEADS * C, HEADS),
                  full(1, HEADS)],
        out_specs=[pl.BlockSpec((HEADS, BLK_N, C), lambda i: (0, i, 0)),
                   pl.BlockSpec((BLK_N, C), lambda i: (i, 0)),
                   pl.BlockSpec((BLK_N, C), lambda i: (i, 0)),
                   pl.BlockSpec((BLK_N, HEADS), lambda i: (i, 0))],
        out_shape=[jax.ShapeDtypeStruct((HEADS, n, C), jnp.float32),
                   jax.ShapeDtypeStruct((n, C), jnp.float32),
                   jax.ShapeDtypeStruct((n, C), jnp.float32),
                   jax.ShapeDtypeStruct((n, HEADS), jnp.float32)],
    )(x, wih0, whh0, b0, wih1, whh1, b1, wgat, asrcw, adstw, awself)


def _k3(src, dst, tabs, tabd):
    e = src.shape[0]
    nchunks = e // CH3
    base_per_w, extra = nchunks // 32, nchunks % 32

    @functools.partial(
        pl.kernel, mesh=_sc_mesh(), **_SC_PARAMS,
        out_type=[jax.ShapeDtypeStruct((e, C), jnp.float32),
                  jax.ShapeDtypeStruct((e, C), jnp.float32)],
        scratch_types=[pltpu.VMEM((CH3 // 128, 1, 128), jnp.int32),
                       pltpu.VMEM((CH3 // 128, 1, 128), jnp.int32),
                       pltpu.VMEM((CH3, C), jnp.float32),
                       pltpu.VMEM((CH3, C), jnp.float32),
                       pltpu.SemaphoreType.DMA,
                       pltpu.SemaphoreType.DMA])
    def k(src_hbm, dst_hbm, tabs_hbm, tabd_hbm, ga_hbm, gd_hbm,
          sidx, didx, gs, gd, sem1, sem2):
        wid = lax.axis_index("s") * 2 + lax.axis_index("c")
        nmine = jnp.where(wid < extra, base_per_w + 1, base_per_w)

        def body(i, _):
            off = (wid + 32 * i) * CH3
            cps = []
            for j in range(CH3 // 128):
                pltpu.sync_copy(src_hbm.at[pl.ds(off + j * 128, 128)],
                                sidx.at[j, 0])
                pltpu.sync_copy(dst_hbm.at[pl.ds(off + j * 128, 128)],
                                didx.at[j, 0])
                cps.append(pltpu.async_copy(
                    tabs_hbm.at[sidx.at[j, 0]],
                    gs.at[pl.ds(j * 128, 128)], sem1))
                cps.append(pltpu.async_copy(
                    tabd_hbm.at[didx.at[j, 0]],
                    gd.at[pl.ds(j * 128, 128)], sem2))
            for cp in cps:
                cp.wait()
            pltpu.sync_copy(gs, ga_hbm.at[pl.ds(off, CH3)])
            pltpu.sync_copy(gd, gd_hbm.at[pl.ds(off, CH3)])
            return 0

        lax.fori_loop(0, nmine, body, 0)

    return k(src, dst, tabs, tabd)


def _k4_body(ga_ref, gd_ref, ea_ref, wtile_ref, t_ref, t0_ref, t1_ref,
             t2_ref, t3_ref):
    rows = ga_ref.shape[0]
    ea = jnp.broadcast_to(ea_ref[...][:, :, None], (rows, 8, C))
    ea = ea.reshape(rows, 128)
    t = jnp.exp(_lrelu(ga_ref[...] + gd_ref[...] + ea * wtile_ref[...]))
    t_ref[...] = t
    t3 = t.reshape(rows, 8, C)
    for h, ref in enumerate((t0_ref, t1_ref, t2_ref, t3_ref)):
        ref[...] = jnp.broadcast_to(
            t3[:, :, h][:, :, None], (rows, 8, C)).reshape(rows, 128)


def _k4(ga, gd, ea8, wvec):
    rows = ga.shape[0]
    grid = rows // BLK_K4
    spec = pl.BlockSpec((BLK_K4, 128), lambda i: (i, 0))
    wtile = jnp.tile(wvec, 32)[None, :]
    shp = jax.ShapeDtypeStruct((rows, 128), jnp.float32)
    return pl.pallas_call(
        _k4_body,
        grid=(grid,),
        in_specs=[spec, spec, pl.BlockSpec((BLK_K4, 8), lambda i: (i, 0)),
                  pl.BlockSpec((1, 128), lambda i: (0, 0))],
        out_specs=[spec] * 5,
        out_shape=[shp] * 5,
    )(ga, gd, ea8, wtile)


def _k5(dst, t16, zeros16):
    e = dst.shape[0]
    n = zeros16.shape[0]
    nchunks = e // CH5
    base_per_w, extra = nchunks // 32, nchunks % 32
    nrchunks = n // NROW
    rbase, rextra = nrchunks // 16, nrchunks % 16

    @functools.partial(
        pl.kernel, mesh=_sc_mesh(), **_SC_PARAMS,
        out_type=jax.ShapeDtypeStruct((2 * n, C), jnp.float32),
        scratch_types=[pltpu.VMEM((CH5 // 128, 1, 128), jnp.int32),
                       pltpu.VMEM((CH5, C), jnp.float32),
                       pltpu.VMEM_SHARED((n, C), jnp.float32)])
    def k(dst_hbm, t_hbm, z_hbm, out_hbm, didx, vv, den_sp):
        c = lax.axis_index("c")
        s = lax.axis_index("s")
        wid = s * 2 + c
        nr = jnp.where(s < rextra, rbase + 1, rbase)

        def zbody(i, _):
            ro = (s + 16 * i) * NROW
            pltpu.sync_copy(z_hbm.at[pl.ds(ro, NROW)],
                            den_sp.at[pl.ds(ro, NROW)])
            return 0

        lax.fori_loop(0, nr, zbody, 0)
        plsc.subcore_barrier()

        nmine = jnp.where(wid < extra, base_per_w + 1, base_per_w)

        def body(i, _):
            off = (wid + 32 * i) * CH5
            pltpu.sync_copy(t_hbm.at[pl.ds(off, CH5)], vv)
            for j in range(CH5 // 128):
                pltpu.sync_copy(dst_hbm.at[pl.ds(off + j * 128, 128)],
                                didx.at[j, 0])
                pltpu.sync_copy(vv.at[pl.ds(j * 128, 128)],
                                den_sp.at[didx.at[j, 0]], add=True)
            return 0

        lax.fori_loop(0, nmine, body, 0)
        plsc.subcore_barrier()

        def wbody(i, _):
            ro = (s + 16 * i) * NROW
            pltpu.sync_copy(den_sp.at[pl.ds(ro, NROW)],
                            out_hbm.at[pl.ds(c * n + ro, NROW)])
            return 0

        lax.fori_loop(0, nr, wbody, 0)

    return k(dst, t16, zeros16)


def _k6(srcsh, dst, xpt, texp0, texp1, texp2, texp3, zeros16):
    e = dst.shape[0]
    n = zeros16.shape[0]
    nchunks = e // CH6
    base_per_w, extra = nchunks // 16, nchunks % 16
    nrchunks = n // NROW
    rbase, rextra = nrchunks // 16, nrchunks % 16

    @functools.partial(
        pl.kernel, mesh=_sc_mesh(), **_SC_PARAMS,
        out_type=jax.ShapeDtypeStruct((HEADS * n, C), jnp.float32),
        scratch_types=[pltpu.VMEM((CH6 // 128, 1, 128), jnp.int32),
                       pltpu.VMEM((CH6 // 128, 1, 128), jnp.int32),
                       pltpu.VMEM((CH6, C), jnp.float32),
                       pltpu.VMEM((CH6, C), jnp.float32),
                       pltpu.VMEM_SHARED((n, C), jnp.float32),
                       pltpu.SemaphoreType.DMA])
    def k(srcsh_hbm, dst_hbm, xpt_hbm, t0_hbm, t1_hbm, t2_hbm, t3_hbm,
          z_hbm, out_hbm, sidx, didx, tv, gv, agg_sp, sem):
        c = lax.axis_index("c")
        s = lax.axis_index("s")
        nr = jnp.where(s < rextra, rbase + 1, rbase)
        nmine = jnp.where(s < extra, base_per_w + 1, base_per_w)

        def round_(h, t_hbm):
            def zbody(i, _):
                ro = (s + 16 * i) * NROW
                pltpu.sync_copy(z_hbm.at[pl.ds(ro, NROW)],
                                agg_sp.at[pl.ds(ro, NROW)])
                return 0

            lax.fori_loop(0, nr, zbody, 0)
            plsc.subcore_barrier()

            def body(i, _):
                off = (s + 16 * i) * CH6
                pltpu.sync_copy(t_hbm.at[pl.ds(off, CH6)], tv)
                cps = []
                for j in range(CH6 // 128):
                    pltpu.sync_copy(
                        srcsh_hbm.at[pl.ds(h * e + off + j * 128, 128)],
                        sidx.at[j, 0])
                    cps.append(pltpu.async_copy(
                        xpt_hbm.at[sidx.at[j, 0]],
                        gv.at[pl.ds(j * 128, 128)], sem))
                for cp in cps:
                    cp.wait()

                def sbody(ei, _):
                    gv[ei, :] = gv[ei, :] * tv[ei, :]
                    return 0

                lax.fori_loop(0, CH6, sbody, 0, unroll=8)
                for j in range(CH6 // 128):
                    pltpu.sync_copy(dst_hbm.at[pl.ds(off + j * 128, 128)],
                                    didx.at[j, 0])
                    pltpu.sync_copy(gv.at[pl.ds(j * 128, 128)],
                                    agg_sp.at[didx.at[j, 0]], add=True)
                return 0

            lax.fori_loop(0, nmine, body, 0)
            plsc.subcore_barrier()

            def wbody(i, _):
                ro = (s + 16 * i) * NROW
                pltpu.sync_copy(agg_sp.at[pl.ds(ro, NROW)],
                                out_hbm.at[pl.ds(h * n + ro, NROW)])
                return 0

            lax.fori_loop(0, nr, wbody, 0)
            plsc.subcore_barrier()

        @pl.when(c == 0)
        def _():
            round_(0, t0_hbm)
            round_(1, t1_hbm)

        @pl.when(c == 1)
        def _():
            round_(2, t2_hbm)
            round_(3, t3_hbm)

    return k(srcsh, dst, xpt, texp0, texp1, texp2, texp3, zeros16)


def _k7_body(agg_ref, den_ref, aself_ref, xp_ref, bias_ref, wfc_ref, bfc_ref,
             out_ref):
    bsz = aself_ref.shape[0]
    agg = jnp.transpose(agg_ref[...], (1, 0, 2)).reshape(bsz, HEADS * C)
    xp = jnp.transpose(xp_ref[...], (1, 0, 2)).reshape(bsz, HEADS * C)
    q = jnp.exp(aself_ref[...])
    den = den_ref[0, :, :HEADS] + den_ref[1, :, :HEADS] + q
    q64 = jnp.broadcast_to(q[:, :, None], (bsz, HEADS, C)).reshape(bsz, HEADS * C)
    d64 = jnp.broadcast_to(den[:, :, None], (bsz, HEADS, C)).reshape(bsz, HEADS * C)
    o = (agg + q64 * xp) / d64 + bias_ref[...]
    o = jnp.where(o > 0, o, jnp.exp(jnp.minimum(o, 0.0)) - 1.0)
    o = jnp.dot(o, wfc_ref[...], preferred_element_type=jnp.float32) + bfc_ref[...]
    out_ref[...] = jnp.maximum(o, 0.0)


def _k7(aggp, den4p, aself4, xp_hm, bias, wfc, bfc):
    n = aself4.shape[0]
    grid = n // BLK_N
    full = lambda *s: pl.BlockSpec(s, lambda i: tuple(0 for _ in s))
    return pl.pallas_call(
        _k7_body,
        grid=(grid,),
        in_specs=[pl.BlockSpec((HEADS, BLK_N, C), lambda i: (0, i, 0)),
                  pl.BlockSpec((2, BLK_N, C), lambda i: (0, i, 0)),
                  pl.BlockSpec((BLK_N, HEADS), lambda i: (i, 0)),
                  pl.BlockSpec((HEADS, BLK_N, C), lambda i: (0, i, 0)),
                  full(1, HEADS * C), full(HEADS * C, NC), full(1, NC)],
        out_specs=pl.BlockSpec((BLK_N, NC), lambda i: (i, 0)),
        out_shape=jax.ShapeDtypeStruct((n, NC), jnp.float32),
    )(aggp, den4p, aself4, xp_hm, bias, wfc, bfc)


def kernel(x, edge_index, edge_attr,
           W_ih0, W_hh0, b_ih0, b_hh0,
           W_ih1, W_hh1, b_ih1, b_hh1,
           W_gat, att_src, att_dst, W_edge, att_edge, bias_gat,
           W_fc, b_fc):
    n = x.shape[0]
    e = edge_attr.shape[0]
    src = edge_index[0].astype(jnp.int32)
    dst = edge_index[1].astype(jnp.int32)

    b0 = (b_ih0 + b_hh0)[None, :]
    b1 = (b_ih1 + b_hh1)[None, :]
    wih0 = W_ih0.T
    whh0 = W_hh0.T
    wih1 = W_ih1.T
    whh1 = W_hh1.T
    hm = (jnp.arange(HEADS * C) // C)[:, None] == jnp.arange(HEADS)[None, :]
    adstw = jnp.where(hm, att_dst.reshape(-1)[:, None], 0.0)
    asrcw = jnp.where(hm, att_src.reshape(-1)[:, None], 0.0)
    wvec = jnp.sum(W_edge.reshape(HEADS, C) * att_edge, axis=1)

    ea_sum = _k1(edge_attr.reshape(e // 128, 128))
    mean_ea = ea_sum / e
    awself = (mean_ea * wvec)[None, :]

    xp_hm, tabs, tabd, aself4 = _k2(
        x, wih0, whh0, b0, wih1, whh1, b1, W_gat, asrcw, adstw, awself)

    ga16, gd16 = _k3(src, dst, tabs, tabd)

    fl = lambda a: a.reshape(e * C // 128, 128)
    t16f, te0, te1, te2, te3 = _k4(fl(ga16), fl(gd16),
                                   edge_attr.reshape(e // 8, 8), wvec)
    t16 = t16f.reshape(e, C)

    zeros16 = jnp.zeros((n, C), jnp.float32)

    den = _k5(dst, t16, zeros16)

    srcsh = (src[None, :] + (jnp.arange(HEADS, dtype=jnp.int32) * n)[:, None]
             ).reshape(-1)
    den, srcsh = lax.optimization_barrier((den, srcsh))
    aggp = _k6(srcsh, dst, xp_hm.reshape(HEADS * n, C),
               te0.reshape(e, C), te1.reshape(e, C),
               te2.reshape(e, C), te3.reshape(e, C), zeros16)

    out = _k7(aggp.reshape(HEADS, n, C), den.reshape(2, n, C), aself4, xp_hm,
              bias_gat[None, :], W_fc, b_fc[None, :])
    return out


# speedup vs baseline: 26.5873x; 1.1956x over previous
import functools

import jax
import jax.numpy as jnp
from jax import lax
from jax.experimental import pallas as pl
from jax.experimental.pallas import tpu as pltpu
from jax.experimental.pallas import tpu_sc as plsc

NS = 16
T = 32
LH = 32
HEADS = 4
C = 16
NC = 8

BLK_N = 2000
BLK_K4 = 1000

CH3 = 2560
CH5 = 1280
CH6 = 640
NROW = 800


def _sig(x):
    return 0.5 * jnp.tanh(0.5 * x) + 0.5


def _lrelu(x):
    return jnp.maximum(x, 0.0) + 0.2 * jnp.minimum(x, 0.0)


def _sc_mesh():
    return plsc.VectorSubcoreMesh(core_axis_name="c", subcore_axis_name="s")


_SC_PARAMS = dict(
    compiler_params=pltpu.CompilerParams(use_tc_tiling_on_sc=False))


def _k1_body(ea_ref, mean_ref):
    mean_ref[...] = jnp.full((1, 128), jnp.sum(ea_ref[...]), jnp.float32)


def _k1(ea_flat2d):
    rows = ea_flat2d.shape[0]
    s = pl.pallas_call(
        _k1_body,
        grid=(1,),
        in_specs=[pl.BlockSpec((rows, 128), lambda i: (0, 0))],
        out_specs=pl.BlockSpec((1, 128), lambda i: (0, 0)),
        out_shape=jax.ShapeDtypeStruct((1, 128), jnp.float32),
    )(ea_flat2d)
    return s[0, 0]


def _k2_body(x_ref, wih0_ref, whh0_ref, b0_ref, wih1_ref, whh1_ref, b1_ref,
             wgat_ref, asrcw_ref, adstw_ref, awself_ref,
             xp_ref, tabs_ref, tabd_ref, aself_ref):
    xb = x_ref[...]
    bsz = xb.shape[0]
    static = xb[:, :NS]
    wih0 = wih0_ref[...]
    whh0 = whh0_ref[...]
    b0 = b0_ref[...]
    wih1 = wih1_ref[...]
    whh1 = whh1_ref[...]
    b1 = b1_ref[...]
    z = jnp.zeros((bsz, LH), jnp.float32)
    h0, c0, h1, c1 = z, z, z, z
    for t in range(T):
        xt = xb[:, NS + t][:, None]
        g0 = xt * wih0 + jnp.dot(h0, whh0, preferred_element_type=jnp.float32) + b0
        i0 = _sig(g0[:, 0:LH])
        f0 = _sig(g0[:, LH:2 * LH])
        gg0 = jnp.tanh(g0[:, 2 * LH:3 * LH])
        o0 = _sig(g0[:, 3 * LH:4 * LH])
        c0 = f0 * c0 + i0 * gg0
        h0 = o0 * jnp.tanh(c0)
        g1 = (jnp.dot(h0, wih1, preferred_element_type=jnp.float32)
              + jnp.dot(h1, whh1, preferred_element_type=jnp.float32) + b1)
        i1 = _sig(g1[:, 0:LH])
        f1 = _sig(g1[:, LH:2 * LH])
        gg1 = jnp.tanh(g1[:, 2 * LH:3 * LH])
        o1 = _sig(g1[:, 3 * LH:4 * LH])
        c1 = f1 * c1 + i1 * gg1
        h1 = o1 * jnp.tanh(c1)
    comb = jnp.concatenate([h1, static], axis=1)
    xp = jnp.dot(comb, wgat_ref[...], preferred_element_type=jnp.float32)
    asrc = jnp.dot(xp, asrcw_ref[...], preferred_element_type=jnp.float32)
    adst = jnp.dot(xp, adstw_ref[...], preferred_element_type=jnp.float32)
    pad = jnp.zeros((bsz, C - HEADS), jnp.float32)
    xp_ref[...] = jnp.transpose(xp.reshape(bsz, HEADS, C), (1, 0, 2))
    tabs_ref[...] = jnp.concatenate([asrc, pad], axis=1)
    tabd_ref[...] = jnp.concatenate([adst, pad], axis=1)
    aself_ref[...] = _lrelu(asrc + adst + awself_ref[...])


def _k2(x, wih0, whh0, b0, wih1, whh1, b1, wgat, asrcw, adstw, awself):
    n = x.shape[0]
    grid = n // BLK_N
    full = lambda *s: pl.BlockSpec(s, lambda i: tuple(0 for _ in s))
    return pl.pallas_call(
        _k2_body,
        grid=(grid,),
        in_specs=[pl.BlockSpec((BLK_N, NS + T), lambda i: (i, 0)),
                  full(1, 128), full(LH, 128), full(1, 128),
                  full(LH, 128), full(LH, 128), full(1, 128),
                  full(NS + LH, HEADS * C),
                  full(HEADS * C, HEADS), full(HEADS * C, HEADS),
                  full(1, HEADS)],
        out_specs=[pl.BlockSpec((HEADS, BLK_N, C), lambda i: (0, i, 0)),
                   pl.BlockSpec((BLK_N, C), lambda i: (i, 0)),
                   pl.BlockSpec((BLK_N, C), lambda i: (i, 0)),
                   pl.BlockSpec((BLK_N, HEADS), lambda i: (i, 0))],
        out_shape=[jax.ShapeDtypeStruct((HEADS, n, C), jnp.float32),
                   jax.ShapeDtypeStruct((n, C), jnp.float32),
                   jax.ShapeDtypeStruct((n, C), jnp.float32),
                   jax.ShapeDtypeStruct((n, HEADS), jnp.float32)],
    )(x, wih0, whh0, b0, wih1, whh1, b1, wgat, asrcw, adstw, awself)


def _k3(src, dst, tabs, tabd):
    e = src.shape[0]
    nchunks = e // CH3
    base_per_w, extra = nchunks // 32, nchunks % 32

    @functools.partial(
        pl.kernel, mesh=_sc_mesh(), **_SC_PARAMS,
        out_type=[jax.ShapeDtypeStruct((e, C), jnp.float32),
                  jax.ShapeDtypeStruct((e, C), jnp.float32)],
        scratch_types=[pltpu.VMEM((CH3 // 128, 1, 128), jnp.int32),
                       pltpu.VMEM((CH3 // 128, 1, 128), jnp.int32),
                       pltpu.VMEM((CH3, C), jnp.float32),
                       pltpu.VMEM((CH3, C), jnp.float32),
                       pltpu.SemaphoreType.DMA,
                       pltpu.SemaphoreType.DMA])
    def k(src_hbm, dst_hbm, tabs_hbm, tabd_hbm, ga_hbm, gd_hbm,
          sidx, didx, gs, gd, sem1, sem2):
        wid = lax.axis_index("s") * 2 + lax.axis_index("c")
        nmine = jnp.where(wid < extra, base_per_w + 1, base_per_w)

        def body(i, _):
            off = (wid + 32 * i) * CH3
            cps = []
            for j in range(CH3 // 128):
                pltpu.sync_copy(src_hbm.at[pl.ds(off + j * 128, 128)],
                                sidx.at[j, 0])
                pltpu.sync_copy(dst_hbm.at[pl.ds(off + j * 128, 128)],
                                didx.at[j, 0])
                cps.append(pltpu.async_copy(
                    tabs_hbm.at[sidx.at[j, 0]],
                    gs.at[pl.ds(j * 128, 128)], sem1))
                cps.append(pltpu.async_copy(
                    tabd_hbm.at[didx.at[j, 0]],
                    gd.at[pl.ds(j * 128, 128)], sem2))
            for cp in cps:
                cp.wait()
            pltpu.sync_copy(gs, ga_hbm.at[pl.ds(off, CH3)])
            pltpu.sync_copy(gd, gd_hbm.at[pl.ds(off, CH3)])
            return 0

        lax.fori_loop(0, nmine, body, 0)

    return k(src, dst, tabs, tabd)


def _k4_body(ga_ref, gd_ref, ea_ref, wtile_ref, q_ref, p_ref,
             t_ref, t0_ref, t1_ref, t2_ref, t3_ref):
    ea = jnp.dot(ea_ref[...], q_ref[...], preferred_element_type=jnp.float32)
    t = jnp.exp(_lrelu(ga_ref[...] + gd_ref[...] + ea * wtile_ref[...]))
    t_ref[...] = t
    for h, ref in enumerate((t0_ref, t1_ref, t2_ref, t3_ref)):
        ref[...] = jnp.dot(t, p_ref[h], preferred_element_type=jnp.float32)


def _k4(ga, gd, ea8, wvec):
    rows = ga.shape[0]
    grid = rows // BLK_K4
    spec = pl.BlockSpec((BLK_K4, 128), lambda i: (i, 0))
    wtile = jnp.tile(wvec, 32)[None, :]
    lane = jnp.arange(128)
    qmat = (lane[None, :] // C == jnp.arange(8)[:, None]).astype(jnp.float32)
    pmat = jnp.stack([
        (((lane[:, None] % C) == h) & (lane[:, None] // C == lane[None, :] // C)
         ).astype(jnp.float32) for h in range(HEADS)])
    full = lambda *s: pl.BlockSpec(s, lambda i: tuple(0 for _ in s))
    shp = jax.ShapeDtypeStruct((rows, 128), jnp.float32)
    return pl.pallas_call(
        _k4_body,
        grid=(grid,),
        in_specs=[spec, spec, pl.BlockSpec((BLK_K4, 8), lambda i: (i, 0)),
                  full(1, 128), full(8, 128), full(HEADS, 128, 128)],
        out_specs=[spec] * 5,
        out_shape=[shp] * 5,
    )(ga, gd, ea8, wtile, qmat, pmat)


def _k5(dst, t16, zeros16):
    e = dst.shape[0]
    n = zeros16.shape[0]
    nchunks = e // CH5
    base_per_w, extra = nchunks // 32, nchunks % 32
    nrchunks = n // NROW
    rbase, rextra = nrchunks // 16, nrchunks % 16

    @functools.partial(
        pl.kernel, mesh=_sc_mesh(), **_SC_PARAMS,
        out_type=jax.ShapeDtypeStruct((2 * n, C), jnp.float32),
        scratch_types=[pltpu.VMEM((CH5 // 128, 1, 128), jnp.int32),
                       pltpu.VMEM((CH5, C), jnp.float32),
                       pltpu.VMEM_SHARED((n, C), jnp.float32)])
    def k(dst_hbm, t_hbm, z_hbm, out_hbm, didx, vv, den_sp):
        c = lax.axis_index("c")
        s = lax.axis_index("s")
        wid = s * 2 + c
        nr = jnp.where(s < rextra, rbase + 1, rbase)

        def zbody(i, _):
            ro = (s + 16 * i) * NROW
            pltpu.sync_copy(z_hbm.at[pl.ds(ro, NROW)],
                            den_sp.at[pl.ds(ro, NROW)])
            return 0

        lax.fori_loop(0, nr, zbody, 0)
        plsc.subcore_barrier()

        nmine = jnp.where(wid < extra, base_per_w + 1, base_per_w)

        def body(i, _):
            off = (wid + 32 * i) * CH5
            pltpu.sync_copy(t_hbm.at[pl.ds(off, CH5)], vv)
            for j in range(CH5 // 128):
                pltpu.sync_copy(dst_hbm.at[pl.ds(off + j * 128, 128)],
                                didx.at[j, 0])
                pltpu.sync_copy(vv.at[pl.ds(j * 128, 128)],
                                den_sp.at[didx.at[j, 0]], add=True)
            return 0

        lax.fori_loop(0, nmine, body, 0)
        plsc.subcore_barrier()

        def wbody(i, _):
            ro = (s + 16 * i) * NROW
            pltpu.sync_copy(den_sp.at[pl.ds(ro, NROW)],
                            out_hbm.at[pl.ds(c * n + ro, NROW)])
            return 0

        lax.fori_loop(0, nr, wbody, 0)

    return k(dst, t16, zeros16)


def _k6(srcsh, dst, xpt, texp0, texp1, texp2, texp3, zeros16):
    e = dst.shape[0]
    n = zeros16.shape[0]
    nchunks = e // CH6
    base_per_w, extra = nchunks // 16, nchunks % 16
    nrchunks = n // NROW
    rbase, rextra = nrchunks // 16, nrchunks % 16

    @functools.partial(
        pl.kernel, mesh=_sc_mesh(), **_SC_PARAMS,
        out_type=jax.ShapeDtypeStruct((HEADS * n, C), jnp.float32),
        scratch_types=[pltpu.VMEM((CH6 // 128, 1, 128), jnp.int32),
                       pltpu.VMEM((CH6 // 128, 1, 128), jnp.int32),
                       pltpu.VMEM((CH6, C), jnp.float32),
                       pltpu.VMEM((CH6, C), jnp.float32),
                       pltpu.VMEM_SHARED((n, C), jnp.float32),
                       pltpu.SemaphoreType.DMA])
    def k(srcsh_hbm, dst_hbm, xpt_hbm, t0_hbm, t1_hbm, t2_hbm, t3_hbm,
          z_hbm, out_hbm, sidx, didx, tv, gv, agg_sp, sem):
        c = lax.axis_index("c")
        s = lax.axis_index("s")
        nr = jnp.where(s < rextra, rbase + 1, rbase)
        nmine = jnp.where(s < extra, base_per_w + 1, base_per_w)

        def round_(h, t_hbm):
            def zbody(i, _):
                ro = (s + 16 * i) * NROW
                pltpu.sync_copy(z_hbm.at[pl.ds(ro, NROW)],
                                agg_sp.at[pl.ds(ro, NROW)])
                return 0

            lax.fori_loop(0, nr, zbody, 0)
            plsc.subcore_barrier()

            def body(i, _):
                off = (s + 16 * i) * CH6
                pltpu.sync_copy(t_hbm.at[pl.ds(off, CH6)], tv)
                cps = []
                for j in range(CH6 // 128):
                    pltpu.sync_copy(
                        srcsh_hbm.at[pl.ds(h * e + off + j * 128, 128)],
                        sidx.at[j, 0])
                    cps.append(pltpu.async_copy(
                        xpt_hbm.at[sidx.at[j, 0]],
                        gv.at[pl.ds(j * 128, 128)], sem))
                for cp in cps:
                    cp.wait()

                def sbody(ei, _):
                    gv[ei, :] = gv[ei, :] * tv[ei, :]
                    return 0

                lax.fori_loop(0, CH6, sbody, 0, unroll=8)
                for j in range(CH6 // 128):
                    pltpu.sync_copy(dst_hbm.at[pl.ds(off + j * 128, 128)],
                                    didx.at[j, 0])
                    pltpu.sync_copy(gv.at[pl.ds(j * 128, 128)],
                                    agg_sp.at[didx.at[j, 0]], add=True)
                return 0

            lax.fori_loop(0, nmine, body, 0)
            plsc.subcore_barrier()

            def wbody(i, _):
                ro = (s + 16 * i) * NROW
                pltpu.sync_copy(agg_sp.at[pl.ds(ro, NROW)],
                                out_hbm.at[pl.ds(h * n + ro, NROW)])
                return 0

            lax.fori_loop(0, nr, wbody, 0)
            plsc.subcore_barrier()

        @pl.when(c == 0)
        def _():
            round_(0, t0_hbm)
            round_(1, t1_hbm)

        @pl.when(c == 1)
        def _():
            round_(2, t2_hbm)
            round_(3, t3_hbm)

    return k(srcsh, dst, xpt, texp0, texp1, texp2, texp3, zeros16)


def _k7_body(agg_ref, den_ref, aself_ref, xp_ref, bias_ref, wfc_ref, bfc_ref,
             out_ref):
    bsz = aself_ref.shape[0]
    agg = jnp.transpose(agg_ref[...], (1, 0, 2)).reshape(bsz, HEADS * C)
    xp = jnp.transpose(xp_ref[...], (1, 0, 2)).reshape(bsz, HEADS * C)
    q = jnp.exp(aself_ref[...])
    den = den_ref[0, :, :HEADS] + den_ref[1, :, :HEADS] + q
    q64 = jnp.broadcast_to(q[:, :, None], (bsz, HEADS, C)).reshape(bsz, HEADS * C)
    d64 = jnp.broadcast_to(den[:, :, None], (bsz, HEADS, C)).reshape(bsz, HEADS * C)
    o = (agg + q64 * xp) / d64 + bias_ref[...]
    o = jnp.where(o > 0, o, jnp.exp(jnp.minimum(o, 0.0)) - 1.0)
    o = jnp.dot(o, wfc_ref[...], preferred_element_type=jnp.float32) + bfc_ref[...]
    out_ref[...] = jnp.maximum(o, 0.0)


def _k7(aggp, den4p, aself4, xp_hm, bias, wfc, bfc):
    n = aself4.shape[0]
    grid = n // BLK_N
    full = lambda *s: pl.BlockSpec(s, lambda i: tuple(0 for _ in s))
    return pl.pallas_call(
        _k7_body,
        grid=(grid,),
        in_specs=[pl.BlockSpec((HEADS, BLK_N, C), lambda i: (0, i, 0)),
                  pl.BlockSpec((2, BLK_N, C), lambda i: (0, i, 0)),
                  pl.BlockSpec((BLK_N, HEADS), lambda i: (i, 0)),
                  pl.BlockSpec((HEADS, BLK_N, C), lambda i: (0, i, 0)),
                  full(1, HEADS * C), full(HEADS * C, NC), full(1, NC)],
        out_specs=pl.BlockSpec((BLK_N, NC), lambda i: (i, 0)),
        out_shape=jax.ShapeDtypeStruct((n, NC), jnp.float32),
    )(aggp, den4p, aself4, xp_hm, bias, wfc, bfc)


def kernel(x, edge_index, edge_attr,
           W_ih0, W_hh0, b_ih0, b_hh0,
           W_ih1, W_hh1, b_ih1, b_hh1,
           W_gat, att_src, att_dst, W_edge, att_edge, bias_gat,
           W_fc, b_fc):
    n = x.shape[0]
    e = edge_attr.shape[0]
    src = edge_index[0].astype(jnp.int32)
    dst = edge_index[1].astype(jnp.int32)

    b0 = (b_ih0 + b_hh0)[None, :]
    b1 = (b_ih1 + b_hh1)[None, :]
    wih0 = W_ih0.T
    whh0 = W_hh0.T
    wih1 = W_ih1.T
    whh1 = W_hh1.T
    hm = (jnp.arange(HEADS * C) // C)[:, None] == jnp.arange(HEADS)[None, :]
    adstw = jnp.where(hm, att_dst.reshape(-1)[:, None], 0.0)
    asrcw = jnp.where(hm, att_src.reshape(-1)[:, None], 0.0)
    wvec = jnp.sum(W_edge.reshape(HEADS, C) * att_edge, axis=1)

    ea_sum = _k1(edge_attr.reshape(e // 128, 128))
    mean_ea = ea_sum / e
    awself = (mean_ea * wvec)[None, :]

    xp_hm, tabs, tabd, aself4 = _k2(
        x, wih0, whh0, b0, wih1, whh1, b1, W_gat, asrcw, adstw, awself)

    ga16, gd16 = _k3(src, dst, tabs, tabd)

    fl = lambda a: a.reshape(e * C // 128, 128)
    t16f, te0, te1, te2, te3 = _k4(fl(ga16), fl(gd16),
                                   edge_attr.reshape(e // 8, 8), wvec)
    t16 = t16f.reshape(e, C)

    zeros16 = jnp.zeros((n, C), jnp.float32)

    den = _k5(dst, t16, zeros16)

    srcsh = (src[None, :] + (jnp.arange(HEADS, dtype=jnp.int32) * n)[:, None]
             ).reshape(-1)
    den, srcsh = lax.optimization_barrier((den, srcsh))
    aggp = _k6(srcsh, dst, xp_hm.reshape(HEADS * n, C),
               te0.reshape(e, C), te1.reshape(e, C),
               te2.reshape(e, C), te3.reshape(e, C), zeros16)

    out = _k7(aggp.reshape(HEADS, n, C), den.reshape(2, n, C), aself4, xp_hm,
              bias_gat[None, :], W_fc, b_fc[None, :])
    return out


# speedup vs baseline: 26.7963x; 1.0079x over previous
import functools

import jax
import jax.numpy as jnp
from jax import lax
from jax.experimental import pallas as pl
from jax.experimental.pallas import tpu as pltpu
from jax.experimental.pallas import tpu_sc as plsc

NS = 16
T = 32
LH = 32
HEADS = 4
C = 16
NC = 8

BLK_N = 2000
BLK_K4 = 2000

CH3 = 2560
CH5 = 1280
CH6 = 640
NROW = 800


def _sig(x):
    return 0.5 * jnp.tanh(0.5 * x) + 0.5


def _lrelu(x):
    return jnp.maximum(x, 0.0) + 0.2 * jnp.minimum(x, 0.0)


def _sc_mesh():
    return plsc.VectorSubcoreMesh(core_axis_name="c", subcore_axis_name="s")


_SC_PARAMS = dict(
    compiler_params=pltpu.CompilerParams(use_tc_tiling_on_sc=False))


def _k1_body(ea_ref, mean_ref):
    mean_ref[...] = jnp.full((1, 128), jnp.sum(ea_ref[...]), jnp.float32)


def _k1(ea_flat2d):
    rows = ea_flat2d.shape[0]
    s = pl.pallas_call(
        _k1_body,
        grid=(1,),
        in_specs=[pl.BlockSpec((rows, 128), lambda i: (0, 0))],
        out_specs=pl.BlockSpec((1, 128), lambda i: (0, 0)),
        out_shape=jax.ShapeDtypeStruct((1, 128), jnp.float32),
    )(ea_flat2d)
    return s[0, 0]


def _k2_body(x_ref, wih0_ref, whh0_ref, b0_ref, wih1_ref, whh1_ref, b1_ref,
             wgat_ref, asrcw_ref, adstw_ref, awself_ref,
             xp_ref, tabs_ref, tabd_ref, aself_ref):
    xb = x_ref[...]
    bsz = xb.shape[0]
    static = xb[:, :NS]
    wih0 = wih0_ref[...]
    whh0 = whh0_ref[...]
    b0 = b0_ref[...]
    wih1 = wih1_ref[...]
    whh1 = whh1_ref[...]
    b1 = b1_ref[...]
    z = jnp.zeros((bsz, LH), jnp.float32)
    h0, c0, h1, c1 = z, z, z, z
    for t in range(T):
        xt = xb[:, NS + t][:, None]
        g0 = xt * wih0 + jnp.dot(h0, whh0, preferred_element_type=jnp.float32) + b0
        i0 = _sig(g0[:, 0:LH])
        f0 = _sig(g0[:, LH:2 * LH])
        gg0 = jnp.tanh(g0[:, 2 * LH:3 * LH])
        o0 = _sig(g0[:, 3 * LH:4 * LH])
        c0 = f0 * c0 + i0 * gg0
        h0 = o0 * jnp.tanh(c0)
        g1 = (jnp.dot(h0, wih1, preferred_element_type=jnp.float32)
              + jnp.dot(h1, whh1, preferred_element_type=jnp.float32) + b1)
        i1 = _sig(g1[:, 0:LH])
        f1 = _sig(g1[:, LH:2 * LH])
        gg1 = jnp.tanh(g1[:, 2 * LH:3 * LH])
        o1 = _sig(g1[:, 3 * LH:4 * LH])
        c1 = f1 * c1 + i1 * gg1
        h1 = o1 * jnp.tanh(c1)
    comb = jnp.concatenate([h1, static], axis=1)
    xp = jnp.dot(comb, wgat_ref[...], preferred_element_type=jnp.float32)
    asrc = jnp.dot(xp, asrcw_ref[...], preferred_element_type=jnp.float32)
    adst = jnp.dot(xp, adstw_ref[...], preferred_element_type=jnp.float32)
    pad = jnp.zeros((bsz, C - HEADS), jnp.float32)
    xp_ref[...] = jnp.transpose(xp.reshape(bsz, HEADS, C), (1, 0, 2))
    tabs_ref[...] = jnp.concatenate([asrc, pad], axis=1)
    tabd_ref[...] = jnp.concatenate([adst, pad], axis=1)
    aself_ref[...] = _lrelu(asrc + adst + awself_ref[...])


def _k2(x, wih0, whh0, b0, wih1, whh1, b1, wgat, asrcw, adstw, awself):
    n = x.shape[0]
    grid = n // BLK_N
    full = lambda *s: pl.BlockSpec(s, lambda i: tuple(0 for _ in s))
    return pl.pallas_call(
        _k2_body,
        grid=(grid,),
        in_specs=[pl.BlockSpec((BLK_N, NS + T), lambda i: (i, 0)),
                  full(1, 128), full(LH, 128), full(1, 128),
                  full(LH, 128), full(LH, 128), full(1, 128),
                  full(NS + LH, HEADS * C),
                  full(HEADS * C, HEADS), full(HEADS * C, HEADS),
                  full(1, HEADS)],
        out_specs=[pl.BlockSpec((HEADS, BLK_N, C), lambda i: (0, i, 0)),
                   pl.BlockSpec((BLK_N, C), lambda i: (i, 0)),
                   pl.BlockSpec((BLK_N, C), lambda i: (i, 0)),
                   pl.BlockSpec((BLK_N, HEADS), lambda i: (i, 0))],
        out_shape=[jax.ShapeDtypeStruct((HEADS, n, C), jnp.float32),
                   jax.ShapeDtypeStruct((n, C), jnp.float32),
                   jax.ShapeDtypeStruct((n, C), jnp.float32),
                   jax.ShapeDtypeStruct((n, HEADS), jnp.float32)],
    )(x, wih0, whh0, b0, wih1, whh1, b1, wgat, asrcw, adstw, awself)


def _k3(src, dst, tabs, tabd):
    e = src.shape[0]
    nchunks = e // CH3
    base_per_w, extra = nchunks // 32, nchunks % 32

    @functools.partial(
        pl.kernel, mesh=_sc_mesh(), **_SC_PARAMS,
        out_type=[jax.ShapeDtypeStruct((e, C), jnp.float32),
                  jax.ShapeDtypeStruct((e, C), jnp.float32)],
        scratch_types=[pltpu.VMEM((CH3 // 128, 1, 128), jnp.int32),
                       pltpu.VMEM((CH3 // 128, 1, 128), jnp.int32),
                       pltpu.VMEM((CH3, C), jnp.float32),
                       pltpu.VMEM((CH3, C), jnp.float32),
                       pltpu.SemaphoreType.DMA,
                       pltpu.SemaphoreType.DMA])
    def k(src_hbm, dst_hbm, tabs_hbm, tabd_hbm, ga_hbm, gd_hbm,
          sidx, didx, gs, gd, sem1, sem2):
        wid = lax.axis_index("s") * 2 + lax.axis_index("c")
        nmine = jnp.where(wid < extra, base_per_w + 1, base_per_w)

        def body(i, _):
            off = (wid + 32 * i) * CH3
            cps = []
            for j in range(CH3 // 128):
                pltpu.sync_copy(src_hbm.at[pl.ds(off + j * 128, 128)],
                                sidx.at[j, 0])
                pltpu.sync_copy(dst_hbm.at[pl.ds(off + j * 128, 128)],
                                didx.at[j, 0])
                cps.append(pltpu.async_copy(
                    tabs_hbm.at[sidx.at[j, 0]],
                    gs.at[pl.ds(j * 128, 128)], sem1))
                cps.append(pltpu.async_copy(
                    tabd_hbm.at[didx.at[j, 0]],
                    gd.at[pl.ds(j * 128, 128)], sem2))
            for cp in cps:
                cp.wait()
            pltpu.sync_copy(gs, ga_hbm.at[pl.ds(off, CH3)])
            pltpu.sync_copy(gd, gd_hbm.at[pl.ds(off, CH3)])
            return 0

        lax.fori_loop(0, nmine, body, 0)

    return k(src, dst, tabs, tabd)


def _k4_body(ga_ref, gd_ref, ea_ref, wtile_ref, q_ref, p_ref,
             t_ref, t0_ref, t1_ref, t2_ref, t3_ref):
    ea = jnp.dot(ea_ref[...], q_ref[...], preferred_element_type=jnp.float32)
    t = jnp.exp(_lrelu(ga_ref[...] + gd_ref[...] + ea * wtile_ref[...]))
    t_ref[...] = t
    for h, ref in enumerate((t0_ref, t1_ref, t2_ref, t3_ref)):
        ref[...] = jnp.dot(t, p_ref[h], preferred_element_type=jnp.float32)


def _k4(ga, gd, ea8, wvec):
    rows = ga.shape[0]
    grid = rows // BLK_K4
    spec = pl.BlockSpec((BLK_K4, 128), lambda i: (i, 0))
    wtile = jnp.tile(wvec, 32)[None, :]
    lane = jnp.arange(128)
    qmat = (lane[None, :] // C == jnp.arange(8)[:, None]).astype(jnp.float32)
    pmat = jnp.stack([
        (((lane[:, None] % C) == h) & (lane[:, None] // C == lane[None, :] // C)
         ).astype(jnp.float32) for h in range(HEADS)])
    full = lambda *s: pl.BlockSpec(s, lambda i: tuple(0 for _ in s))
    shp = jax.ShapeDtypeStruct((rows, 128), jnp.float32)
    return pl.pallas_call(
        _k4_body,
        grid=(grid,),
        in_specs=[spec, spec, pl.BlockSpec((BLK_K4, 8), lambda i: (i, 0)),
                  full(1, 128), full(8, 128), full(HEADS, 128, 128)],
        out_specs=[spec] * 5,
        out_shape=[shp] * 5,
    )(ga, gd, ea8, wtile, qmat, pmat)


def _k5(dst, t16, zeros16):
    e = dst.shape[0]
    n = zeros16.shape[0]
    nchunks = e // CH5
    base_per_w, extra = nchunks // 32, nchunks % 32
    nrchunks = n // NROW
    rbase, rextra = nrchunks // 16, nrchunks % 16

    @functools.partial(
        pl.kernel, mesh=_sc_mesh(), **_SC_PARAMS,
        out_type=jax.ShapeDtypeStruct((2 * n, C), jnp.float32),
        scratch_types=[pltpu.VMEM((CH5 // 128, 1, 128), jnp.int32),
                       pltpu.VMEM((CH5, C), jnp.float32),
                       pltpu.VMEM_SHARED((n, C), jnp.float32)])
    def k(dst_hbm, t_hbm, z_hbm, out_hbm, didx, vv, den_sp):
        c = lax.axis_index("c")
        s = lax.axis_index("s")
        wid = s * 2 + c
        nr = jnp.where(s < rextra, rbase + 1, rbase)

        def zbody(i, _):
            ro = (s + 16 * i) * NROW
            pltpu.sync_copy(z_hbm.at[pl.ds(ro, NROW)],
                            den_sp.at[pl.ds(ro, NROW)])
            return 0

        lax.fori_loop(0, nr, zbody, 0)
        plsc.subcore_barrier()

        nmine = jnp.where(wid < extra, base_per_w + 1, base_per_w)

        def body(i, _):
            off = (wid + 32 * i) * CH5
            pltpu.sync_copy(t_hbm.at[pl.ds(off, CH5)], vv)
            for j in range(CH5 // 128):
                pltpu.sync_copy(dst_hbm.at[pl.ds(off + j * 128, 128)],
                                didx.at[j, 0])
                pltpu.sync_copy(vv.at[pl.ds(j * 128, 128)],
                                den_sp.at[didx.at[j, 0]], add=True)
            return 0

        lax.fori_loop(0, nmine, body, 0)
        plsc.subcore_barrier()

        def wbody(i, _):
            ro = (s + 16 * i) * NROW
            pltpu.sync_copy(den_sp.at[pl.ds(ro, NROW)],
                            out_hbm.at[pl.ds(c * n + ro, NROW)])
            return 0

        lax.fori_loop(0, nr, wbody, 0)

    return k(dst, t16, zeros16)


def _k6(srcsh, dst, xpt, texp0, texp1, texp2, texp3, zeros16):
    e = dst.shape[0]
    n = zeros16.shape[0]
    nchunks = e // CH6
    base_per_w, extra = nchunks // 16, nchunks % 16
    nrchunks = n // NROW
    rbase, rextra = nrchunks // 16, nrchunks % 16

    @functools.partial(
        pl.kernel, mesh=_sc_mesh(), **_SC_PARAMS,
        out_type=jax.ShapeDtypeStruct((HEADS * n, C), jnp.float32),
        scratch_types=[pltpu.VMEM((CH6 // 128, 1, 128), jnp.int32),
                       pltpu.VMEM((CH6 // 128, 1, 128), jnp.int32),
                       pltpu.VMEM((CH6, C), jnp.float32),
                       pltpu.VMEM((CH6, C), jnp.float32),
                       pltpu.VMEM_SHARED((n, C), jnp.float32),
                       pltpu.SemaphoreType.DMA])
    def k(srcsh_hbm, dst_hbm, xpt_hbm, t0_hbm, t1_hbm, t2_hbm, t3_hbm,
          z_hbm, out_hbm, sidx, didx, tv, gv, agg_sp, sem):
        c = lax.axis_index("c")
        s = lax.axis_index("s")
        nr = jnp.where(s < rextra, rbase + 1, rbase)
        nmine = jnp.where(s < extra, base_per_w + 1, base_per_w)

        def round_(h, t_hbm):
            def zbody(i, _):
                ro = (s + 16 * i) * NROW
                pltpu.sync_copy(z_hbm.at[pl.ds(ro, NROW)],
                                agg_sp.at[pl.ds(ro, NROW)])
                return 0

            lax.fori_loop(0, nr, zbody, 0)
            plsc.subcore_barrier()

            def body(i, _):
                off = (s + 16 * i) * CH6
                pltpu.sync_copy(t_hbm.at[pl.ds(off, CH6)], tv)
                cps = []
                for j in range(CH6 // 128):
                    pltpu.sync_copy(
                        srcsh_hbm.at[pl.ds(h * e + off + j * 128, 128)],
                        sidx.at[j, 0])
                    cps.append(pltpu.async_copy(
                        xpt_hbm.at[sidx.at[j, 0]],
                        gv.at[pl.ds(j * 128, 128)], sem))
                for cp in cps:
                    cp.wait()

                def sbody(ei, _):
                    gv[ei, :] = gv[ei, :] * tv[ei, :]
                    return 0

                lax.fori_loop(0, CH6, sbody, 0, unroll=8)
                for j in range(CH6 // 128):
                    pltpu.sync_copy(dst_hbm.at[pl.ds(off + j * 128, 128)],
                                    didx.at[j, 0])
                    pltpu.sync_copy(gv.at[pl.ds(j * 128, 128)],
                                    agg_sp.at[didx.at[j, 0]], add=True)
                return 0

            lax.fori_loop(0, nmine, body, 0)
            plsc.subcore_barrier()

            def wbody(i, _):
                ro = (s + 16 * i) * NROW
                pltpu.sync_copy(agg_sp.at[pl.ds(ro, NROW)],
                                out_hbm.at[pl.ds(h * n + ro, NROW)])
                return 0

            lax.fori_loop(0, nr, wbody, 0)
            plsc.subcore_barrier()

        @pl.when(c == 0)
        def _():
            round_(0, t0_hbm)
            round_(1, t1_hbm)

        @pl.when(c == 1)
        def _():
            round_(2, t2_hbm)
            round_(3, t3_hbm)

    return k(srcsh, dst, xpt, texp0, texp1, texp2, texp3, zeros16)


def _k7_body(agg_ref, den_ref, aself_ref, xp_ref, bias_ref, wfc_ref, bfc_ref,
             out_ref):
    bsz = aself_ref.shape[0]
    agg = jnp.transpose(agg_ref[...], (1, 0, 2)).reshape(bsz, HEADS * C)
    xp = jnp.transpose(xp_ref[...], (1, 0, 2)).reshape(bsz, HEADS * C)
    q = jnp.exp(aself_ref[...])
    den = den_ref[0, :, :HEADS] + den_ref[1, :, :HEADS] + q
    q64 = jnp.broadcast_to(q[:, :, None], (bsz, HEADS, C)).reshape(bsz, HEADS * C)
    d64 = jnp.broadcast_to(den[:, :, None], (bsz, HEADS, C)).reshape(bsz, HEADS * C)
    o = (agg + q64 * xp) / d64 + bias_ref[...]
    o = jnp.where(o > 0, o, jnp.exp(jnp.minimum(o, 0.0)) - 1.0)
    o = jnp.dot(o, wfc_ref[...], preferred_element_type=jnp.float32) + bfc_ref[...]
    out_ref[...] = jnp.maximum(o, 0.0)


def _k7(aggp, den4p, aself4, xp_hm, bias, wfc, bfc):
    n = aself4.shape[0]
    grid = n // BLK_N
    full = lambda *s: pl.BlockSpec(s, lambda i: tuple(0 for _ in s))
    return pl.pallas_call(
        _k7_body,
        grid=(grid,),
        in_specs=[pl.BlockSpec((HEADS, BLK_N, C), lambda i: (0, i, 0)),
                  pl.BlockSpec((2, BLK_N, C), lambda i: (0, i, 0)),
                  pl.BlockSpec((BLK_N, HEADS), lambda i: (i, 0)),
                  pl.BlockSpec((HEADS, BLK_N, C), lambda i: (0, i, 0)),
                  full(1, HEADS * C), full(HEADS * C, NC), full(1, NC)],
        out_specs=pl.BlockSpec((BLK_N, NC), lambda i: (i, 0)),
        out_shape=jax.ShapeDtypeStruct((n, NC), jnp.float32),
    )(aggp, den4p, aself4, xp_hm, bias, wfc, bfc)


def kernel(x, edge_index, edge_attr,
           W_ih0, W_hh0, b_ih0, b_hh0,
           W_ih1, W_hh1, b_ih1, b_hh1,
           W_gat, att_src, att_dst, W_edge, att_edge, bias_gat,
           W_fc, b_fc):
    n = x.shape[0]
    e = edge_attr.shape[0]
    src = edge_index[0].astype(jnp.int32)
    dst = edge_index[1].astype(jnp.int32)

    b0 = (b_ih0 + b_hh0)[None, :]
    b1 = (b_ih1 + b_hh1)[None, :]
    wih0 = W_ih0.T
    whh0 = W_hh0.T
    wih1 = W_ih1.T
    whh1 = W_hh1.T
    hm = (jnp.arange(HEADS * C) // C)[:, None] == jnp.arange(HEADS)[None, :]
    adstw = jnp.where(hm, att_dst.reshape(-1)[:, None], 0.0)
    asrcw = jnp.where(hm, att_src.reshape(-1)[:, None], 0.0)
    wvec = jnp.sum(W_edge.reshape(HEADS, C) * att_edge, axis=1)

    ea_sum = _k1(edge_attr.reshape(e // 128, 128))
    mean_ea = ea_sum / e
    awself = (mean_ea * wvec)[None, :]

    xp_hm, tabs, tabd, aself4 = _k2(
        x, wih0, whh0, b0, wih1, whh1, b1, W_gat, asrcw, adstw, awself)

    ga16, gd16 = _k3(src, dst, tabs, tabd)

    fl = lambda a: a.reshape(e * C // 128, 128)
    t16f, te0, te1, te2, te3 = _k4(fl(ga16), fl(gd16),
                                   edge_attr.reshape(e // 8, 8), wvec)
    t16 = t16f.reshape(e, C)

    zeros16 = jnp.zeros((n, C), jnp.float32)

    den = _k5(dst, t16, zeros16)

    srcsh = (src[None, :] + (jnp.arange(HEADS, dtype=jnp.int32) * n)[:, None]
             ).reshape(-1)
    den, srcsh = lax.optimization_barrier((den, srcsh))
    aggp = _k6(srcsh, dst, xp_hm.reshape(HEADS * n, C),
               te0.reshape(e, C), te1.reshape(e, C),
               te2.reshape(e, C), te3.reshape(e, C), zeros16)

    out = _k7(aggp.reshape(HEADS, n, C), den.reshape(2, n, C), aself4, xp_hm,
              bias_gat[None, :], W_fc, b_fc[None, :])
    return out
